# baseline (speedup 1.0000x reference)
import functools

import jax
import jax.numpy as jnp
from jax import lax
from jax.experimental import pallas as pl
from jax.experimental.pallas import tpu as pltpu
from jax.experimental.pallas import tpu_sc as plsc

VOCAB = 100277
DIM = 20
DIMH = DIM // 2
B = 16384
LANES = 16


def _sc_info():
    try:
        info = plsc.get_sparse_core_info()
        return info.num_cores, info.num_subcores
    except Exception:
        return 2, 16


def _make_gather(num_cores, num_subcores):
    nw = num_cores * num_subcores
    bpw = B // nw
    nchunk = bpw // LANES
    mesh = plsc.VectorSubcoreMesh(core_axis_name="c", subcore_axis_name="s")

    @functools.partial(
        pl.kernel,
        mesh=mesh,
        out_type=jax.ShapeDtypeStruct((DIMH, B), jnp.int32),
        scratch_types=[
            pltpu.VMEM((bpw,), jnp.int32),
            pltpu.VMEM((DIMH * bpw,), jnp.int32),
            pltpu.VMEM((DIMH * bpw,), jnp.int32),
            pltpu.SemaphoreType.DMA,
            pltpu.SemaphoreType.DMA,
        ],
    )
    def gather(tflat_hbm, idx_hbm, out_hbm, idx_v, ilist_v, rows_v, sem, sem2):
        wid = lax.axis_index("s") * num_cores + lax.axis_index("c")
        base = wid * bpw
        pltpu.sync_copy(idx_hbm.at[pl.ds(base, bpw)], idx_v)
        for c in range(nchunk):
            v = idx_v[pl.ds(c * LANES, LANES)]
            for d in range(DIMH):
                ilist_v[pl.ds(d * bpw + c * LANES, LANES)] = v + d * VOCAB
        pltpu.async_copy(tflat_hbm.at[ilist_v], rows_v, sem).wait()
        copies = [
            pltpu.async_copy(
                rows_v.at[pl.ds(d * bpw, bpw)],
                out_hbm.at[d, pl.ds(base, bpw)],
                sem2,
            )
            for d in range(DIMH)
        ]
        for cp in copies:
            cp.wait()

    return gather


def _linear_body(rows_ref, we_ref, wo_ref, b_ref, out_ref):
    x = rows_ref[...]
    lo = lax.bitcast_convert_type(jnp.left_shift(x, 16), jnp.float32)
    hi = lax.bitcast_convert_type(
        jnp.bitwise_and(x, jnp.int32(-65536)), jnp.float32
    )
    dn = (((1,), (0,)), ((), ()))
    out_ref[...] = (
        lax.dot_general(we_ref[...], lo, dn, preferred_element_type=jnp.float32)
        + lax.dot_general(wo_ref[...], hi, dn, preferred_element_type=jnp.float32)
        + b_ref[...]
    )


def kernel(input, table, W, b):
    ids = input[:, -1].astype(jnp.int32)
    tT = table.T
    evi = lax.bitcast_convert_type(
        tT[0::2].astype(jnp.bfloat16), jnp.uint16
    ).astype(jnp.uint32)
    odi = lax.bitcast_convert_type(
        tT[1::2].astype(jnp.bfloat16), jnp.uint16
    ).astype(jnp.uint32)
    tpk = lax.bitcast_convert_type(
        evi | (odi << jnp.uint32(16)), jnp.int32
    ).reshape(-1)
    nc, ns = _sc_info()
    rows_p = _make_gather(nc, ns)(tpk, ids)
    out_t = pl.pallas_call(
        _linear_body,
        out_shape=jax.ShapeDtypeStruct((DIM, B), jnp.float32),
    )(rows_p, W[:, 0::2], W[:, 1::2], b.reshape(DIM, 1))
    return out_t.T


# speedup vs baseline: 1.7966x; 1.7966x over previous
import functools

import jax
import jax.numpy as jnp
from jax import lax
from jax.experimental import pallas as pl
from jax.experimental.pallas import tpu as pltpu
from jax.experimental.pallas import tpu_sc as plsc

VOCAB = 100277
DIM = 20
B = 16384
LANES = 16


def _sc_info():
    try:
        info = plsc.get_sparse_core_info()
        return info.num_cores, info.num_subcores
    except Exception:
        return 2, 16


def _make_gather(num_cores, num_subcores):
    nw = num_cores * num_subcores
    bpw = B // nw
    nchunk = bpw // LANES
    mesh = plsc.VectorSubcoreMesh(core_axis_name="c", subcore_axis_name="s")

    @functools.partial(
        pl.kernel,
        mesh=mesh,
        out_type=jax.ShapeDtypeStruct((DIM, B), jnp.float32),
        scratch_types=[
            pltpu.VMEM((bpw,), jnp.int32),
            pltpu.VMEM((DIM * bpw,), jnp.int32),
            pltpu.VMEM((DIM * bpw,), jnp.float32),
            pltpu.SemaphoreType.DMA,
            pltpu.SemaphoreType.DMA,
        ],
    )
    def gather(tflat_hbm, idx_hbm, out_hbm, idx_v, ilist_v, rows_v, sem, sem2):
        wid = lax.axis_index("s") * num_cores + lax.axis_index("c")
        base = wid * bpw
        pltpu.sync_copy(idx_hbm.at[pl.ds(base, bpw)], idx_v)
        for c in range(nchunk):
            v = idx_v[pl.ds(c * LANES, LANES)]
            for d in range(DIM):
                ilist_v[pl.ds(d * bpw + c * LANES, LANES)] = v + d * VOCAB
        pltpu.async_copy(tflat_hbm.at[ilist_v], rows_v, sem).wait()
        copies = [
            pltpu.async_copy(
                rows_v.at[pl.ds(d * bpw, bpw)],
                out_hbm.at[d, pl.ds(base, bpw)],
                sem2,
            )
            for d in range(DIM)
        ]
        for cp in copies:
            cp.wait()

    return gather


def _linear_body(rows_ref, w_ref, b_ref, out_ref):
    out_ref[...] = (
        lax.dot_general(
            w_ref[...],
            rows_ref[...],
            (((1,), (0,)), ((), ())),
            preferred_element_type=jnp.float32,
        )
        + b_ref[...]
    )


def kernel(input, table, W, b):
    ids = input[:, -1].astype(jnp.int32)
    tflat = table.T.reshape(-1)
    nc, ns = _sc_info()
    rows_t = _make_gather(nc, ns)(tflat, ids)
    out_t = pl.pallas_call(
        _linear_body,
        out_shape=jax.ShapeDtypeStruct((DIM, B), jnp.float32),
    )(rows_t, W, b.reshape(DIM, 1))
    return out_t.T
